# Initial kernel scaffold; baseline (speedup 1.0000x reference)
import jax
import jax.numpy as jnp
from jax.experimental import pallas as pl


def kernel(latents, codebook):
    raise NotImplementedError("write your pallas kernel here")


# speedup vs baseline: 1.8006x; 1.8006x over previous
import functools

import jax
import jax.numpy as jnp
from jax import lax
from jax.experimental import pallas as pl
from jax.experimental.pallas import tpu as pltpu
from jax.experimental.pallas import tpu_sc as plsc

_NUM_CODES = 8192
_CODE_DIM = 64
_ROWS = 512

_NC = 2
_NS = 16
_NW = _NC * _NS
_CHUNK = 128


def _argmin_kernel(x_ref, xsq_ref, cb_ref, csq_ref, codes_ref):
    x2 = (x_ref[...] * 2.0).astype(jnp.bfloat16)
    cb = cb_ref[...].astype(jnp.bfloat16)
    dots2 = lax.dot_general(x2, cb, (((1,), (1,)), ((), ())),
                            preferred_element_type=jnp.float32)
    dist = xsq_ref[...] - dots2 + csq_ref[...]
    codes_ref[0, 0, :] = jnp.argmin(dist, axis=1).astype(jnp.int32)


def _tc_codes(flat, x_sq, codebook, c_sq):
    n = flat.shape[0]
    n_tiles = n // _ROWS
    codes3 = pl.pallas_call(
        _argmin_kernel,
        grid=(n_tiles,),
        in_specs=[
            pl.BlockSpec((_ROWS, _CODE_DIM), lambda i: (i, 0)),
            pl.BlockSpec((_ROWS, 1), lambda i: (i, 0)),
            pl.BlockSpec((_NUM_CODES, _CODE_DIM), lambda i: (0, 0)),
            pl.BlockSpec((1, _NUM_CODES), lambda i: (0, 0)),
        ],
        out_specs=pl.BlockSpec((1, 1, _ROWS), lambda i: (i, 0, 0)),
        out_shape=jax.ShapeDtypeStruct((n_tiles, 1, _ROWS), jnp.int32),
    )(flat, x_sq, codebook, c_sq)
    return codes3.reshape(n)


def _sc_gather(codebook, codes_flat):
    n = codes_flat.shape[0]
    n_chunks = n // (_NW * _CHUNK)
    codes3 = codes_flat.reshape(_NW, n_chunks, _CHUNK)
    mesh = plsc.VectorSubcoreMesh(core_axis_name="c", subcore_axis_name="s")

    @functools.partial(
        pl.kernel,
        mesh=mesh,
        out_type=jax.ShapeDtypeStruct((_NW, n_chunks, _CHUNK, _CODE_DIM),
                                      jnp.float32),
        scratch_types=[
            pltpu.VMEM((n_chunks, _CHUNK), jnp.int32),
            pltpu.VMEM((n_chunks, _CHUNK, _CODE_DIM), jnp.float32),
            pltpu.SemaphoreType.DMA,
        ],
        compiler_params=pltpu.CompilerParams(use_tc_tiling_on_sc=False),
    )
    def gather_k(cb_hbm, idx_hbm, out_hbm, idx_v, rows_v, sem):
        wid = lax.axis_index("s") * _NC + lax.axis_index("c")
        pltpu.sync_copy(idx_hbm.at[wid], idx_v)
        copies = [
            pltpu.async_copy(cb_hbm.at[idx_v.at[j]], rows_v.at[j], sem)
            for j in range(n_chunks)
        ]
        for cp in copies:
            cp.wait()
        pltpu.sync_copy(rows_v, out_hbm.at[wid])

    out = gather_k(codebook, codes3)
    return out.reshape(n, _CODE_DIM)


def kernel(latents, codebook):
    b, s, d = latents.shape
    flat = latents.reshape(-1, d)
    x_sq = jnp.sum(flat * flat, axis=1, keepdims=True)
    c_sq = jnp.sum(codebook * codebook, axis=1)[None, :]
    codes_flat = _tc_codes(flat, x_sq, codebook, c_sq)
    quantized = _sc_gather(codebook, codes_flat).reshape(latents.shape)
    codes = codes_flat.reshape(b, s)
    return quantized, codes
